# baseline (speedup 1.0000x reference)
import jax
import jax.numpy as jnp
from jax import lax
from jax.experimental import pallas as pl
from jax.experimental.pallas import tpu as pltpu
from jax.experimental.pallas import tpu_sc as plsc

N = 1_000_000
D = 27
NCLS = 10
K = 3

RB = 8192
GRID = (N + RB - 1) // RB
NPAD = RB * GRID
NWORK = 32
RPW = NPAD // NWORK
GROUPS = RPW // 16
BIG = 1e19
INT_MAX = 2**31 - 1


def _colmax_body(f_ref, out_ref, ft_ref):
    i = pl.program_id(0)
    x = f_ref[...]
    rows = lax.broadcasted_iota(jnp.int32, (RB, D), 0) + i * RB
    a = jnp.where(rows < N, jnp.abs(x), 0.0)
    part = jnp.max(a, axis=0, keepdims=True)

    @pl.when(i == 0)
    def _():
        out_ref[...] = part

    @pl.when(i > 0)
    def _():
        out_ref[...] = jnp.maximum(out_ref[...], part)

    eye = (lax.broadcasted_iota(jnp.int32, (D, D), 0)
           == lax.broadcasted_iota(jnp.int32, (D, D), 1)).astype(jnp.float32)
    ft_ref[...] = lax.dot_general(
        eye, x, (((1,), (1,)), ((), ())),
        preferred_element_type=jnp.float32)


def _dist2_body(w_ref, si_ref, ft_ref, out_ref):
    i = pl.program_id(0)
    x = ft_ref[...]
    t = x * w_ref[...] - si_ref[...]
    d2 = jnp.sum(t * t, axis=0, keepdims=True)
    cols = lax.broadcasted_iota(jnp.int32, (1, RB), 1) + i * RB
    d2 = jnp.where(cols < N, d2, BIG)
    out_ref[...] = d2.reshape(RB)


def _sc_topk_body(d2_hbm, vals_hbm, idx_hbm, buf, vbuf, ibuf):
    c = lax.axis_index("c")
    s = lax.axis_index("s")
    wid = s * 2 + c
    base = wid * RPW
    pltpu.sync_copy(d2_hbm.at[pl.ds(base, RPW)], buf)
    iota = lax.iota(jnp.int32, 16)
    inf = jnp.full((16,), jnp.inf, jnp.float32)
    zero = jnp.zeros((16,), jnp.int32)

    def body(g, carry):
        m0, m1, m2, i0, i1, i2 = carry
        off = g * 16
        v = buf[pl.ds(off, 16)]
        iv = iota + (base + off)
        c0 = v < m0
        c1 = v < m1
        c2 = v < m2
        nm2 = jnp.where(c1, m1, jnp.where(c2, v, m2))
        ni2 = jnp.where(c1, i1, jnp.where(c2, iv, i2))
        nm1 = jnp.where(c0, m0, jnp.where(c1, v, m1))
        ni1 = jnp.where(c0, i0, jnp.where(c1, iv, i1))
        nm0 = jnp.where(c0, v, m0)
        ni0 = jnp.where(c0, iv, i0)
        return nm0, nm1, nm2, ni0, ni1, ni2

    m0, m1, m2, i0, i1, i2 = lax.fori_loop(
        0, GROUPS, body, (inf, inf, inf, zero, zero, zero))
    vbuf[pl.ds(0, 16)] = m0
    vbuf[pl.ds(16, 16)] = m1
    vbuf[pl.ds(32, 16)] = m2
    ibuf[pl.ds(0, 16)] = i0
    ibuf[pl.ds(16, 16)] = i1
    ibuf[pl.ds(32, 16)] = i2
    pltpu.sync_copy(vbuf, vals_hbm.at[wid])
    pltpu.sync_copy(ibuf, idx_hbm.at[wid])


def _merge_body(vals_ref, idx_ref, kd_ref, ki_ref):
    v = vals_ref[...]
    ii = idx_ref[...]
    lanes = lax.broadcasted_iota(jnp.int32, (1, 8), 1)
    kd = jnp.zeros((1, 8), jnp.float32)
    ki = jnp.zeros((1, 8), jnp.int32)
    for r in range(K):
        g = jnp.min(v)
        gi = jnp.min(jnp.where(v == g, ii, INT_MAX))
        v = jnp.where((v == g) & (ii == gi), jnp.inf, v)
        kd = jnp.where(lanes == r, jnp.sqrt(g), kd)
        ki = jnp.where(lanes == r, gi, ki)
    kd_ref[...] = kd
    ki_ref[...] = ki


def _vote_body(ki_ref, kd_ref, labels_ref, out_ref, lrows, sem):
    for k in range(K):
        idx = ki_ref[0, k]
        pltpu.make_async_copy(
            labels_ref.at[pl.ds(idx, 1), :],
            lrows.at[pl.ds(k, 1), :],
            sem,
        ).start()
    for k in range(K):
        pltpu.make_async_copy(
            labels_ref.at[pl.ds(0, 1), :],
            lrows.at[pl.ds(k, 1), :],
            sem,
        ).wait()
    d0 = kd_ref[0, 0]
    d1 = kd_ref[0, 1]
    d2 = kd_ref[0, 2]
    w0 = 1.0 / jnp.where(d0 == 0.0, 1.0, d0)
    w1 = 1.0 / jnp.where(d1 == 0.0, 1.0, d1)
    w2 = 1.0 / jnp.where(d2 == 0.0, 1.0, d2)
    r0 = lrows[0:1, :]
    r1 = lrows[1:2, :]
    r2 = lrows[2:3, :]
    acc = r0 * w0 + r1 * w1 + r2 * w2
    mx = jnp.max(acc)
    lane = lax.broadcasted_iota(jnp.int32, (1, NCLS), 1)
    am = jnp.min(jnp.where(acc == mx, lane, INT_MAX))
    onehot = jnp.where(lane == am, 1.0, 0.0).astype(jnp.float32)
    mind = jnp.minimum(d0, jnp.minimum(d1, d2))
    out_ref[...] = jnp.where(mind == 0.0, r0, onehot)


_COLMAX = pl.pallas_call(
    _colmax_body,
    grid=(GRID,),
    in_specs=[pl.BlockSpec((RB, D), lambda i: (i, 0))],
    out_specs=[
        pl.BlockSpec((1, D), lambda i: (0, 0)),
        pl.BlockSpec((D, RB), lambda i: (0, i)),
    ],
    out_shape=[
        jax.ShapeDtypeStruct((1, D), jnp.float32),
        jax.ShapeDtypeStruct((D, NPAD), jnp.float32),
    ],
)

_DIST2 = pl.pallas_call(
    _dist2_body,
    grid=(GRID,),
    in_specs=[
        pl.BlockSpec((D, 1), lambda i: (0, 0)),
        pl.BlockSpec((D, 1), lambda i: (0, 0)),
        pl.BlockSpec((D, RB), lambda i: (0, i)),
    ],
    out_specs=pl.BlockSpec((RB,), lambda i: (i,)),
    out_shape=jax.ShapeDtypeStruct((NPAD,), jnp.float32),
)

_MERGE = pl.pallas_call(
    _merge_body,
    in_specs=[
        pl.BlockSpec((NWORK, 48), lambda: (0, 0)),
        pl.BlockSpec((NWORK, 48), lambda: (0, 0)),
    ],
    out_specs=[
        pl.BlockSpec((1, 8), lambda: (0, 0)),
        pl.BlockSpec((1, 8), lambda: (0, 0)),
    ],
    out_shape=[
        jax.ShapeDtypeStruct((1, 8), jnp.float32),
        jax.ShapeDtypeStruct((1, 8), jnp.int32),
    ],
)

_VOTE = pl.pallas_call(
    _vote_body,
    in_specs=[
        pl.BlockSpec(memory_space=pltpu.SMEM),
        pl.BlockSpec(memory_space=pltpu.SMEM),
        pl.BlockSpec(memory_space=pl.ANY),
    ],
    out_specs=pl.BlockSpec((1, NCLS), lambda: (0, 0)),
    out_shape=jax.ShapeDtypeStruct((1, NCLS), jnp.float32),
    scratch_shapes=[
        pltpu.VMEM((8, NCLS), jnp.float32),
        pltpu.SemaphoreType.DMA,
    ],
)

_SC_TOPK_CACHE = []


def _sc_topk_call():
    if not _SC_TOPK_CACHE:
        _SC_TOPK_CACHE.append(pl.kernel(
            _sc_topk_body,
            out_type=[
                jax.ShapeDtypeStruct((NWORK, 48), jnp.float32),
                jax.ShapeDtypeStruct((NWORK, 48), jnp.int32),
            ],
            mesh=plsc.VectorSubcoreMesh(
                core_axis_name="c", subcore_axis_name="s"),
            scratch_types=[
                pltpu.VMEM((RPW,), jnp.float32),
                pltpu.VMEM((48,), jnp.float32),
                pltpu.VMEM((48,), jnp.int32),
            ],
        ))
    return _SC_TOPK_CACHE[0]


def kernel(input, training_data_features, training_data_labels):
    f = training_data_features
    scale, ft = _COLMAX(f)
    scale_col = scale.reshape(D, 1)
    w_col = jnp.where(scale_col == 0.0, 0.0,
                      1.0 / jnp.where(scale_col == 0.0, 1.0, scale_col))
    si_col = input.reshape(D, 1) * w_col
    d2 = _DIST2(w_col, si_col, ft)
    vals, idxs = _sc_topk_call()(d2)
    kd, ki = _MERGE(vals, idxs)
    res = _VOTE(ki, kd, training_data_labels)
    return (kd[0, :K], res.reshape(NCLS))


# speedup vs baseline: 1.0078x; 1.0078x over previous
import jax
import jax.numpy as jnp
from jax import lax
from jax.experimental import pallas as pl
from jax.experimental.pallas import tpu as pltpu
from jax.experimental.pallas import tpu_sc as plsc

N = 1_000_000
D = 27
NCLS = 10
K = 3

NWORK = 32
CRC = 2000
CWORDS = CRC * D
NCHUNK = N // CRC
KMAIN = NCHUNK // NWORK
TAILW = NCHUNK - KMAIN * NWORK
GPC = CRC // 16
INT_MAX = 2**31 - 1


def _wid():
    return lax.axis_index("s") * 2 + lax.axis_index("c")


def _sc_colmax_body(f_hbm, out_hbm, buf, vbuf):
    wid = _wid()
    iota27 = lax.iota(jnp.int32, 16) * D

    def process(chunk_id, accs):
        pltpu.sync_copy(f_hbm.at[pl.ds(chunk_id * CWORDS, CWORDS)], buf)

        def grp(g, accs):
            base = g * (16 * D)
            return tuple(
                jnp.maximum(accs[d],
                            jnp.abs(plsc.load_gather(buf, [iota27 + (base + d)])))
                for d in range(D))

        return lax.fori_loop(0, GPC, grp, accs)

    accs = tuple(jnp.zeros((16,), jnp.float32) for _ in range(D))
    accs = lax.fori_loop(0, KMAIN, lambda k, a: process(wid + NWORK * k, a),
                         accs)
    for d in range(D):
        vbuf[pl.ds(16 * d, 16)] = accs[d]

    @pl.when(wid < TAILW)
    def _():
        accs2 = tuple(vbuf[pl.ds(16 * d, 16)] for d in range(D))
        accs2 = process(KMAIN * NWORK + wid, accs2)
        for d in range(D):
            vbuf[pl.ds(16 * d, 16)] = accs2[d]

    pltpu.sync_copy(vbuf, out_hbm.at[wid])


def _finalize_body(cm_ref, inp_ref, w_ref, si_ref):
    cm = cm_ref[...]
    inp = inp_ref[...]
    lane = lax.broadcasted_iota(jnp.int32, (1, 16 * D), 1)
    grp = lane // 16
    wsp = jnp.zeros((1, 16 * D), jnp.float32)
    ssp = jnp.zeros((1, 16 * D), jnp.float32)
    for d in range(D):
        md = jnp.max(cm[:, 16 * d:16 * d + 16])
        wd = jnp.where(md == 0.0, 0.0, 1.0 / jnp.where(md == 0.0, 1.0, md))
        sd = inp[0, d] * wd
        wsp = jnp.where(grp == d, wd, wsp)
        ssp = jnp.where(grp == d, sd, ssp)
    w_ref[...] = wsp.reshape(16 * D)
    si_ref[...] = ssp.reshape(16 * D)


def _sc_dist_body(f_hbm, w_hbm, si_hbm, vals_hbm, idx_hbm,
                  buf, wbuf, sibuf, vbuf, ibuf):
    wid = _wid()
    pltpu.sync_copy(w_hbm, wbuf)
    pltpu.sync_copy(si_hbm, sibuf)
    iota27 = lax.iota(jnp.int32, 16) * D
    iota = lax.iota(jnp.int32, 16)
    wv = tuple(wbuf[pl.ds(16 * d, 16)] for d in range(D))
    sv = tuple(sibuf[pl.ds(16 * d, 16)] for d in range(D))

    def process(chunk_id, st):
        pltpu.sync_copy(f_hbm.at[pl.ds(chunk_id * CWORDS, CWORDS)], buf)
        row0 = chunk_id * CRC

        def grp(g, st):
            m0, m1, m2, i0, i1, i2 = st
            base = g * (16 * D)
            acc = jnp.zeros((16,), jnp.float32)
            for d in range(D):
                v = plsc.load_gather(buf, [iota27 + (base + d)])
                t = v * wv[d] - sv[d]
                acc = acc + t * t
            iv = iota + (row0 + g * 16)
            c0 = acc < m0
            c1 = acc < m1
            c2 = acc < m2
            nm2 = jnp.where(c1, m1, jnp.where(c2, acc, m2))
            ni2 = jnp.where(c1, i1, jnp.where(c2, iv, i2))
            nm1 = jnp.where(c0, m0, jnp.where(c1, acc, m1))
            ni1 = jnp.where(c0, i0, jnp.where(c1, iv, i1))
            nm0 = jnp.where(c0, acc, m0)
            ni0 = jnp.where(c0, iv, i0)
            return nm0, nm1, nm2, ni0, ni1, ni2

        return lax.fori_loop(0, GPC, grp, st)

    inf = jnp.full((16,), jnp.inf, jnp.float32)
    zero = jnp.zeros((16,), jnp.int32)
    st = (inf, inf, inf, zero, zero, zero)
    st = lax.fori_loop(0, KMAIN, lambda k, s: process(wid + NWORK * k, s), st)
    m0, m1, m2, i0, i1, i2 = st
    vbuf[pl.ds(0, 16)] = m0
    vbuf[pl.ds(16, 16)] = m1
    vbuf[pl.ds(32, 16)] = m2
    ibuf[pl.ds(0, 16)] = i0
    ibuf[pl.ds(16, 16)] = i1
    ibuf[pl.ds(32, 16)] = i2

    @pl.when(wid < TAILW)
    def _():
        st2 = (vbuf[pl.ds(0, 16)], vbuf[pl.ds(16, 16)], vbuf[pl.ds(32, 16)],
               ibuf[pl.ds(0, 16)], ibuf[pl.ds(16, 16)], ibuf[pl.ds(32, 16)])
        n0, n1, n2, j0, j1, j2 = process(KMAIN * NWORK + wid, st2)
        vbuf[pl.ds(0, 16)] = n0
        vbuf[pl.ds(16, 16)] = n1
        vbuf[pl.ds(32, 16)] = n2
        ibuf[pl.ds(0, 16)] = j0
        ibuf[pl.ds(16, 16)] = j1
        ibuf[pl.ds(32, 16)] = j2

    pltpu.sync_copy(vbuf, vals_hbm.at[wid])
    pltpu.sync_copy(ibuf, idx_hbm.at[wid])


def _merge_body(vals_ref, idx_ref, kd_ref, ki_ref):
    v = vals_ref[...]
    ii = idx_ref[...]
    lanes = lax.broadcasted_iota(jnp.int32, (1, 8), 1)
    kd = jnp.zeros((1, 8), jnp.float32)
    ki = jnp.zeros((1, 8), jnp.int32)
    for r in range(K):
        g = jnp.min(v)
        gi = jnp.min(jnp.where(v == g, ii, INT_MAX))
        v = jnp.where((v == g) & (ii == gi), jnp.inf, v)
        kd = jnp.where(lanes == r, jnp.sqrt(g), kd)
        ki = jnp.where(lanes == r, gi, ki)
    kd_ref[...] = kd
    ki_ref[...] = ki


def _vote_body(ki_ref, kd_ref, labels_ref, out_ref, lrows, sem):
    for k in range(K):
        idx = ki_ref[0, k]
        pltpu.make_async_copy(
            labels_ref.at[pl.ds(idx, 1), :],
            lrows.at[pl.ds(k, 1), :],
            sem,
        ).start()
    for k in range(K):
        pltpu.make_async_copy(
            labels_ref.at[pl.ds(0, 1), :],
            lrows.at[pl.ds(k, 1), :],
            sem,
        ).wait()
    d0 = kd_ref[0, 0]
    d1 = kd_ref[0, 1]
    d2 = kd_ref[0, 2]
    w0 = 1.0 / jnp.where(d0 == 0.0, 1.0, d0)
    w1 = 1.0 / jnp.where(d1 == 0.0, 1.0, d1)
    w2 = 1.0 / jnp.where(d2 == 0.0, 1.0, d2)
    r0 = lrows[0:1, :]
    r1 = lrows[1:2, :]
    r2 = lrows[2:3, :]
    acc = r0 * w0 + r1 * w1 + r2 * w2
    mx = jnp.max(acc)
    lane = lax.broadcasted_iota(jnp.int32, (1, NCLS), 1)
    am = jnp.min(jnp.where(acc == mx, lane, INT_MAX))
    onehot = jnp.where(lane == am, 1.0, 0.0).astype(jnp.float32)
    mind = jnp.minimum(d0, jnp.minimum(d1, d2))
    out_ref[...] = jnp.where(mind == 0.0, r0, onehot)


_FINALIZE = pl.pallas_call(
    _finalize_body,
    in_specs=[
        pl.BlockSpec((NWORK, 16 * D), lambda: (0, 0)),
        pl.BlockSpec((1, NWORK), lambda: (0, 0)),
    ],
    out_specs=[
        pl.BlockSpec(memory_space=pltpu.VMEM),
        pl.BlockSpec(memory_space=pltpu.VMEM),
    ],
    out_shape=[
        jax.ShapeDtypeStruct((16 * D,), jnp.float32),
        jax.ShapeDtypeStruct((16 * D,), jnp.float32),
    ],
)

_MERGE = pl.pallas_call(
    _merge_body,
    in_specs=[
        pl.BlockSpec((NWORK, 48), lambda: (0, 0)),
        pl.BlockSpec((NWORK, 48), lambda: (0, 0)),
    ],
    out_specs=[
        pl.BlockSpec((1, 8), lambda: (0, 0)),
        pl.BlockSpec((1, 8), lambda: (0, 0)),
    ],
    out_shape=[
        jax.ShapeDtypeStruct((1, 8), jnp.float32),
        jax.ShapeDtypeStruct((1, 8), jnp.int32),
    ],
)

_VOTE = pl.pallas_call(
    _vote_body,
    in_specs=[
        pl.BlockSpec(memory_space=pltpu.SMEM),
        pl.BlockSpec(memory_space=pltpu.SMEM),
        pl.BlockSpec(memory_space=pl.ANY),
    ],
    out_specs=pl.BlockSpec((1, NCLS), lambda: (0, 0)),
    out_shape=jax.ShapeDtypeStruct((1, NCLS), jnp.float32),
    scratch_shapes=[
        pltpu.VMEM((8, NCLS), jnp.float32),
        pltpu.SemaphoreType.DMA,
    ],
)

_SC_CACHE = {}


def _sc_calls():
    if not _SC_CACHE:
        mesh = plsc.VectorSubcoreMesh(core_axis_name="c", subcore_axis_name="s")
        params = pltpu.CompilerParams(needs_layout_passes=False)
        _SC_CACHE["colmax"] = pl.kernel(
            _sc_colmax_body,
            compiler_params=params,
            out_type=jax.ShapeDtypeStruct((NWORK, 16 * D), jnp.float32),
            mesh=mesh,
            scratch_types=[
                pltpu.VMEM((CWORDS,), jnp.float32),
                pltpu.VMEM((16 * D,), jnp.float32),
            ],
        )
        _SC_CACHE["dist"] = pl.kernel(
            _sc_dist_body,
            compiler_params=params,
            out_type=[
                jax.ShapeDtypeStruct((NWORK, 48), jnp.float32),
                jax.ShapeDtypeStruct((NWORK, 48), jnp.int32),
            ],
            mesh=mesh,
            scratch_types=[
                pltpu.VMEM((CWORDS,), jnp.float32),
                pltpu.VMEM((16 * D,), jnp.float32),
                pltpu.VMEM((16 * D,), jnp.float32),
                pltpu.VMEM((48,), jnp.float32),
                pltpu.VMEM((48,), jnp.int32),
            ],
        )
    return _SC_CACHE


def kernel(input, training_data_features, training_data_labels):
    f_flat = jnp.reshape(training_data_features, (N * D,))
    sc = _sc_calls()
    cm = sc["colmax"](f_flat)
    inp_pad = jnp.zeros((1, NWORK), jnp.float32).at[0, :D].set(
        input.reshape(D))
    w, si = _FINALIZE(cm, inp_pad)
    vals, idxs = sc["dist"](f_flat, w, si)
    kd, ki = _MERGE(vals, idxs)
    res = _VOTE(ki, kd, training_data_labels)
    return (kd[0, :K], res.reshape(NCLS))


# speedup vs baseline: 1.0078x; 1.0000x over previous
import jax
import jax.numpy as jnp
from jax import lax
from jax.experimental import pallas as pl
from jax.experimental.pallas import tpu as pltpu
from jax.experimental.pallas import tpu_sc as plsc

N = 1_000_000
D = 27
NCLS = 10
K = 3

NWORK = 32
CRC = 2000
CWORDS = CRC * D
NCHUNK = N // CRC
KMAIN = NCHUNK // NWORK
TAILW = NCHUNK - KMAIN * NWORK
GPC = CRC // 16
INT_MAX = 2**31 - 1


def _wid():
    return lax.axis_index("s") * 2 + lax.axis_index("c")


def _sc_colmax_body(f_hbm, out_hbm, buf, vbuf):
    wid = _wid()
    iota27 = lax.iota(jnp.int32, 16) * D

    def process(chunk_id, accs):
        pltpu.sync_copy(f_hbm.at[pl.ds(chunk_id * CWORDS, CWORDS)], buf)

        def grp(g, accs):
            base = g * (16 * D)
            return tuple(
                jnp.maximum(accs[d],
                            jnp.abs(plsc.load_gather(buf, [iota27 + (base + d)])))
                for d in range(D))

        return lax.fori_loop(0, GPC, grp, accs)

    accs = tuple(jnp.zeros((16,), jnp.float32) for _ in range(D))
    accs = lax.fori_loop(0, KMAIN, lambda k, a: process(wid + NWORK * k, a),
                         accs)
    for d in range(D):
        vbuf[pl.ds(16 * d, 16)] = accs[d]

    @pl.when(wid < TAILW)
    def _():
        accs2 = tuple(vbuf[pl.ds(16 * d, 16)] for d in range(D))
        accs2 = process(KMAIN * NWORK + wid, accs2)
        for d in range(D):
            vbuf[pl.ds(16 * d, 16)] = accs2[d]

    pltpu.sync_copy(vbuf, out_hbm.at[wid])


def _finalize_body(cm_ref, inp_ref, w_ref, si_ref):
    cm = cm_ref[...]
    inp = inp_ref[...]
    lane = lax.broadcasted_iota(jnp.int32, (1, 16 * D), 1)
    grp = lane // 16
    wsp = jnp.zeros((1, 16 * D), jnp.float32)
    ssp = jnp.zeros((1, 16 * D), jnp.float32)
    for d in range(D):
        md = jnp.max(cm[:, 16 * d:16 * d + 16])
        wd = jnp.where(md == 0.0, 0.0, 1.0 / jnp.where(md == 0.0, 1.0, md))
        sd = inp[0, d] * wd
        wsp = jnp.where(grp == d, wd, wsp)
        ssp = jnp.where(grp == d, sd, ssp)
    w_ref[...] = wsp.reshape(16 * D)
    si_ref[...] = ssp.reshape(16 * D)


def _sc_dist_body(f_hbm, w_hbm, si_hbm, vals_hbm, idx_hbm,
                  buf, wbuf, sibuf, vbuf, ibuf):
    wid = _wid()
    pltpu.sync_copy(w_hbm, wbuf)
    pltpu.sync_copy(si_hbm, sibuf)
    iota27 = lax.iota(jnp.int32, 16) * D
    iota = lax.iota(jnp.int32, 16)
    wv = tuple(wbuf[pl.ds(16 * d, 16)] for d in range(D))
    sv = tuple(sibuf[pl.ds(16 * d, 16)] for d in range(D))

    def process(chunk_id, st):
        pltpu.sync_copy(f_hbm.at[pl.ds(chunk_id * CWORDS, CWORDS)], buf)
        row0 = chunk_id * CRC

        def grp(g, st):
            m0, m1, m2, i0, i1, i2 = st
            base = g * (16 * D)
            acc = jnp.zeros((16,), jnp.float32)
            for d in range(D):
                v = plsc.load_gather(buf, [iota27 + (base + d)])
                t = v * wv[d] - sv[d]
                acc = acc + t * t
            iv = iota + (row0 + g * 16)
            c0 = acc < m0
            c1 = acc < m1
            c2 = acc < m2
            nm2 = jnp.where(c1, m1, jnp.where(c2, acc, m2))
            ni2 = jnp.where(c1, i1, jnp.where(c2, iv, i2))
            nm1 = jnp.where(c0, m0, jnp.where(c1, acc, m1))
            ni1 = jnp.where(c0, i0, jnp.where(c1, iv, i1))
            nm0 = jnp.where(c0, acc, m0)
            ni0 = jnp.where(c0, iv, i0)
            return nm0, nm1, nm2, ni0, ni1, ni2

        return lax.fori_loop(0, GPC, grp, st)

    inf = jnp.full((16,), jnp.inf, jnp.float32)
    zero = jnp.zeros((16,), jnp.int32)
    st = (inf, inf, inf, zero, zero, zero)
    st = lax.fori_loop(0, KMAIN, lambda k, s: process(wid + NWORK * k, s), st)
    m0, m1, m2, i0, i1, i2 = st
    vbuf[pl.ds(0, 16)] = m0
    vbuf[pl.ds(16, 16)] = m1
    vbuf[pl.ds(32, 16)] = m2
    ibuf[pl.ds(0, 16)] = i0
    ibuf[pl.ds(16, 16)] = i1
    ibuf[pl.ds(32, 16)] = i2

    @pl.when(wid < TAILW)
    def _():
        st2 = (vbuf[pl.ds(0, 16)], vbuf[pl.ds(16, 16)], vbuf[pl.ds(32, 16)],
               ibuf[pl.ds(0, 16)], ibuf[pl.ds(16, 16)], ibuf[pl.ds(32, 16)])
        n0, n1, n2, j0, j1, j2 = process(KMAIN * NWORK + wid, st2)
        vbuf[pl.ds(0, 16)] = n0
        vbuf[pl.ds(16, 16)] = n1
        vbuf[pl.ds(32, 16)] = n2
        ibuf[pl.ds(0, 16)] = j0
        ibuf[pl.ds(16, 16)] = j1
        ibuf[pl.ds(32, 16)] = j2

    pltpu.sync_copy(vbuf, vals_hbm.at[wid])
    pltpu.sync_copy(ibuf, idx_hbm.at[wid])


def _merge_body(vals_ref, idx_ref, kd_ref, ki_ref):
    v = vals_ref[...]
    ii = idx_ref[...]
    lanes = lax.broadcasted_iota(jnp.int32, (1, 8), 1)
    kd = jnp.zeros((1, 8), jnp.float32)
    ki = jnp.zeros((1, 8), jnp.int32)
    for r in range(K):
        g = jnp.min(v)
        gi = jnp.min(jnp.where(v == g, ii, INT_MAX))
        v = jnp.where((v == g) & (ii == gi), jnp.inf, v)
        kd = jnp.where(lanes == r, jnp.sqrt(g), kd)
        ki = jnp.where(lanes == r, gi, ki)
    kd_ref[...] = kd
    ki_ref[...] = ki


def _vote_body(ki_ref, kd_ref, labels_ref, out_ref, lrows, sem):
    for k in range(K):
        idx = ki_ref[0, k]
        pltpu.make_async_copy(
            labels_ref.at[pl.ds(idx, 1), :],
            lrows.at[pl.ds(k, 1), :],
            sem,
        ).start()
    for k in range(K):
        pltpu.make_async_copy(
            labels_ref.at[pl.ds(0, 1), :],
            lrows.at[pl.ds(k, 1), :],
            sem,
        ).wait()
    d0 = kd_ref[0, 0]
    d1 = kd_ref[0, 1]
    d2 = kd_ref[0, 2]
    w0 = 1.0 / jnp.where(d0 == 0.0, 1.0, d0)
    w1 = 1.0 / jnp.where(d1 == 0.0, 1.0, d1)
    w2 = 1.0 / jnp.where(d2 == 0.0, 1.0, d2)
    r0 = lrows[0:1, :]
    r1 = lrows[1:2, :]
    r2 = lrows[2:3, :]
    acc = r0 * w0 + r1 * w1 + r2 * w2
    mx = jnp.max(acc)
    lane = lax.broadcasted_iota(jnp.int32, (1, NCLS), 1)
    am = jnp.min(jnp.where(acc == mx, lane, INT_MAX))
    onehot = jnp.where(lane == am, 1.0, 0.0).astype(jnp.float32)
    mind = jnp.minimum(d0, jnp.minimum(d1, d2))
    out_ref[...] = jnp.where(mind == 0.0, r0, onehot)


_FINALIZE = pl.pallas_call(
    _finalize_body,
    in_specs=[
        pl.BlockSpec((NWORK, 16 * D), lambda: (0, 0)),
        pl.BlockSpec((1, NWORK), lambda: (0, 0)),
    ],
    out_specs=[
        pl.BlockSpec(memory_space=pltpu.VMEM),
        pl.BlockSpec(memory_space=pltpu.VMEM),
    ],
    out_shape=[
        jax.ShapeDtypeStruct((16 * D,), jnp.float32),
        jax.ShapeDtypeStruct((16 * D,), jnp.float32),
    ],
)

_MERGE = pl.pallas_call(
    _merge_body,
    in_specs=[
        pl.BlockSpec((NWORK, 48), lambda: (0, 0)),
        pl.BlockSpec((NWORK, 48), lambda: (0, 0)),
    ],
    out_specs=[
        pl.BlockSpec((1, 8), lambda: (0, 0)),
        pl.BlockSpec((1, 8), lambda: (0, 0)),
    ],
    out_shape=[
        jax.ShapeDtypeStruct((1, 8), jnp.float32),
        jax.ShapeDtypeStruct((1, 8), jnp.int32),
    ],
)

_VOTE = pl.pallas_call(
    _vote_body,
    in_specs=[
        pl.BlockSpec(memory_space=pltpu.SMEM),
        pl.BlockSpec(memory_space=pltpu.SMEM),
        pl.BlockSpec(memory_space=pl.ANY),
    ],
    out_specs=pl.BlockSpec((1, NCLS), lambda: (0, 0)),
    out_shape=jax.ShapeDtypeStruct((1, NCLS), jnp.float32),
    scratch_shapes=[
        pltpu.VMEM((8, NCLS), jnp.float32),
        pltpu.SemaphoreType.DMA,
    ],
)

_SC_CACHE = {}


def _sc_calls():
    if not _SC_CACHE:
        mesh = plsc.VectorSubcoreMesh(core_axis_name="c", subcore_axis_name="s")
        params = pltpu.CompilerParams(needs_layout_passes=False)
        _SC_CACHE["colmax"] = pl.kernel(
            _sc_colmax_body,
            compiler_params=params,
            out_type=jax.ShapeDtypeStruct((NWORK, 16 * D), jnp.float32),
            mesh=mesh,
            scratch_types=[
                pltpu.VMEM((CWORDS,), jnp.float32),
                pltpu.VMEM((16 * D,), jnp.float32),
            ],
        )
        _SC_CACHE["dist"] = pl.kernel(
            _sc_dist_body,
            compiler_params=params,
            out_type=[
                jax.ShapeDtypeStruct((NWORK, 48), jnp.float32),
                jax.ShapeDtypeStruct((NWORK, 48), jnp.int32),
            ],
            mesh=mesh,
            scratch_types=[
                pltpu.VMEM((CWORDS,), jnp.float32),
                pltpu.VMEM((16 * D,), jnp.float32),
                pltpu.VMEM((16 * D,), jnp.float32),
                pltpu.VMEM((48,), jnp.float32),
                pltpu.VMEM((48,), jnp.int32),
            ],
        )
    return _SC_CACHE


def kernel(input, training_data_features, training_data_labels):
    f_flat = lax.optimization_barrier(
        jnp.reshape(training_data_features, (N * D,)))
    sc = _sc_calls()
    cm = sc["colmax"](f_flat)
    inp_pad = jnp.zeros((1, NWORK), jnp.float32).at[0, :D].set(
        input.reshape(D))
    w, si = _FINALIZE(cm, inp_pad)
    vals, idxs = sc["dist"](f_flat, w, si)
    kd, ki = _MERGE(vals, idxs)
    res = _VOTE(ki, kd, training_data_labels)
    return (kd[0, :K], res.reshape(NCLS))
